# baseline (speedup 1.0000x reference)
import jax
import jax.numpy as jnp
from jax import lax
from jax.experimental import pallas as pl
from jax.experimental.pallas import tpu as pltpu
from jax.experimental.pallas import tpu_sc as plsc

B = 16384
L = 20
D = 64
YD = 4
OD = D + YD
BL = B * L

NC = 2
NS = 16
NW = NC * NS
ROWS_W = BL // NW
BW = B // NW
CHUNK = 256
NCHUNK = ROWS_W // CHUNK
LANES = 16
NGRP = CHUNK // LANES

_DIV20_MUL = 52429
_DIV20_SHIFT = 20

_GDN = lax.GatherDimensionNumbers(
    offset_dims=(), collapsed_slice_dims=(0,), start_index_map=(0,))


def _lane_gather(v, idx):
  return lax.gather(v, idx[:, None], _GDN, (1,),
                    mode=lax.GatherScatterMode.PROMISE_IN_BOUNDS)


def _bcast(v, i):
  return _lane_gather(v, jnp.full((LANES,), i, jnp.int32))


def _body(table, idxf, yawf, vc,
          feat_out, mask_out,
          idx_v, yaw_v, vc_v, mout_v, out_v, emb_v,
          gsem0, gsem1, osem0, osem1, bsem):
  wid = lax.axis_index("s") * NC + lax.axis_index("c")
  w0 = wid * ROWS_W

  iota = lax.iota(jnp.int32, LANES)
  yaw_row_pat = lax.shift_right_logical(iota, 2)
  yaw_col_pat = D + lax.bitwise_and(iota, 3)

  gsems = (gsem0, gsem1)
  osems = (osem0, osem1)

  pltpu.make_async_copy(idxf.at[pl.ds(w0, ROWS_W)], idx_v, bsem).start()
  pltpu.make_async_copy(yawf.at[pl.ds(w0 * YD, ROWS_W * YD)], yaw_v,
                        bsem).start()
  pltpu.make_async_copy(vc.at[pl.ds(wid * BW, BW)], vc_v, bsem).start()
  pltpu.make_async_copy(idxf.at[pl.ds(w0, ROWS_W)], idx_v, bsem).wait()
  pltpu.make_async_copy(yawf.at[pl.ds(w0 * YD, ROWS_W * YD)], yaw_v,
                        bsem).wait()
  pltpu.make_async_copy(vc.at[pl.ds(wid * BW, BW)], vc_v, bsem).wait()

  def gather_copy(c, s):
    return pltpu.make_async_copy(
        table.at[idx_v.at[pl.ds(c * CHUNK, CHUNK)]],
        emb_v.at[s], gsems[s])

  def output_copies(c, s):
    base = w0 + c * CHUNK
    return [
        pltpu.make_async_copy(
            out_v.at[s], feat_out.at[pl.ds(base, CHUNK)], osems[s]),
        pltpu.make_async_copy(
            mout_v.at[s], mask_out.at[pl.ds(base, CHUNK)], osems[s]),
    ]

  def compute(c, s):
    gather_copy(c, s).wait()

    def grp_body(g, _):
      r0 = c * CHUNK + g * LANES
      rows = r0 + iota
      bloc = lax.shift_right_logical(rows * _DIV20_MUL, _DIV20_SHIFT)
      lpos = rows - bloc * L
      cnt16 = plsc.load_gather(vc_v, [bloc])
      valid = lpos < cnt16
      m16 = jnp.where(valid, 1.0, 0.0).astype(jnp.float32)
      mout_v[s, pl.ds(g * LANES, LANES)] = jnp.where(valid, 0, 1).astype(jnp.int32)
      o0 = g * (LANES * OD)
      for i in range(LANES):
        m = _bcast(m16, i)
        for k in range(D // LANES):
          v = emb_v[s, g * LANES + i, pl.ds(k * LANES, LANES)]
          out_v[s, g * LANES + i, pl.ds(k * LANES, LANES)] = v * m
      for jj in range(4):
        my = _lane_gather(m16, 4 * jj + yaw_row_pat)
        v = yaw_v[pl.ds(r0 * YD + jj * LANES, LANES)]
        rows2 = g * LANES + 4 * jj + yaw_row_pat
        plsc.store_scatter(out_v.at[s], [rows2, yaw_col_pat], v * my)
      return 0

    lax.fori_loop(0, NGRP, grp_body, 0)
    for cp in output_copies(c, s):
      cp.start()

  gather_copy(0, 0).start()
  gather_copy(1, 1).start()

  def loop_body(c2, _):
    a = 2 * c2

    @pl.when(c2 > 0)
    def _():
      for cp in output_copies(a - 2, 0):
        cp.wait()
    compute(a, 0)

    @pl.when(c2 < NCHUNK // 2 - 1)
    def _():
      gather_copy(a + 2, 0).start()

    @pl.when(c2 > 0)
    def _():
      for cp in output_copies(a - 1, 1):
        cp.wait()
    compute(a + 1, 1)

    @pl.when(c2 < NCHUNK // 2 - 1)
    def _():
      gather_copy(a + 3, 1).start()
    return 0

  lax.fori_loop(0, NCHUNK // 2, loop_body, 0)
  for cp in output_copies(NCHUNK - 2, 0):
    cp.wait()
  for cp in output_copies(NCHUNK - 1, 1):
    cp.wait()


@jax.jit
def _run(table, idxf, yawf, vc):
  mesh = plsc.VectorSubcoreMesh(core_axis_name="c", subcore_axis_name="s",
                                num_cores=NC, num_subcores=NS)
  f = pl.kernel(
      _body,
      out_type=(
          jax.ShapeDtypeStruct((BL, OD), jnp.float32),
          jax.ShapeDtypeStruct((BL,), jnp.int32),
      ),
      mesh=mesh,
      compiler_params=pltpu.CompilerParams(use_tc_tiling_on_sc=False,
                                           needs_layout_passes=False),
      scratch_types=[
          pltpu.VMEM((ROWS_W,), jnp.int32),
          pltpu.VMEM((ROWS_W * YD,), jnp.float32),
          pltpu.VMEM((BW,), jnp.int32),
          pltpu.VMEM((2, CHUNK), jnp.int32),
          pltpu.VMEM((2, CHUNK, OD), jnp.float32),
          pltpu.VMEM((2, CHUNK, D), jnp.float32),
          pltpu.SemaphoreType.DMA,
          pltpu.SemaphoreType.DMA,
          pltpu.SemaphoreType.DMA,
          pltpu.SemaphoreType.DMA,
          pltpu.SemaphoreType.DMA,
      ],
  )
  return f(table, idxf, yawf, vc)


def kernel(indices, yaw_bits, valid_counts, table):
  idxf = indices.reshape(-1)
  yawf = yaw_bits.reshape(-1)
  feat, mask_i = _run(table, idxf, yawf, valid_counts)
  features = feat.reshape(B, L, OD)
  mask = mask_i.reshape(B, L).astype(bool)
  return features, mask


# speedup vs baseline: 1.0716x; 1.0716x over previous
import jax
import jax.numpy as jnp
from jax import lax
from jax.experimental import pallas as pl
from jax.experimental.pallas import tpu as pltpu
from jax.experimental.pallas import tpu_sc as plsc

B = 16384
L = 20
D = 64
YD = 4
OD = D + YD
BL = B * L

NC = 2
NS = 16
NW = NC * NS
ROWS_W = BL // NW
BW = B // NW
CHUNK = 256
NCHUNK = ROWS_W // CHUNK
LANES = 16
NGRP = CHUNK // LANES

_DIV20_MUL = 52429
_DIV20_SHIFT = 20

_GDN = lax.GatherDimensionNumbers(
    offset_dims=(), collapsed_slice_dims=(0,), start_index_map=(0,))


def _lane_gather(v, idx):
  return lax.gather(v, idx[:, None], _GDN, (1,),
                    mode=lax.GatherScatterMode.PROMISE_IN_BOUNDS)


def _bcast(v, i):
  return _lane_gather(v, jnp.full((LANES,), i, jnp.int32))


def _body(table, idxf, yawf, vc,
          feat_out, mask_out,
          idx_v, yaw_v, vc_v, mout_v, out_v, emb_v,
          gsem0, gsem1, osem0, osem1, bsem):
  wid = lax.axis_index("s") * NC + lax.axis_index("c")
  w0 = wid * ROWS_W

  iota = lax.iota(jnp.int32, LANES)
  yaw_row_pat = lax.shift_right_logical(iota, 2)
  yaw_col_pat = D + lax.bitwise_and(iota, 3)

  gsems = (gsem0, gsem1)
  osems = (osem0, osem1)

  pltpu.make_async_copy(idxf.at[pl.ds(w0, ROWS_W)], idx_v, bsem).start()
  pltpu.make_async_copy(yawf.at[pl.ds(w0 * YD, ROWS_W * YD)], yaw_v,
                        bsem).start()
  pltpu.make_async_copy(vc.at[pl.ds(wid * BW, BW)], vc_v, bsem).start()
  pltpu.make_async_copy(idxf.at[pl.ds(w0, ROWS_W)], idx_v, bsem).wait()
  pltpu.make_async_copy(yawf.at[pl.ds(w0 * YD, ROWS_W * YD)], yaw_v,
                        bsem).wait()
  pltpu.make_async_copy(vc.at[pl.ds(wid * BW, BW)], vc_v, bsem).wait()

  def gather_copy(c, s):
    return pltpu.make_async_copy(
        table.at[idx_v.at[pl.ds(c * CHUNK, CHUNK)]],
        emb_v.at[s], gsems[s])

  def output_copies(c, s):
    base = w0 + c * CHUNK
    return [
        pltpu.make_async_copy(
            out_v.at[s], feat_out.at[pl.ds(base, CHUNK)], osems[s]),
        pltpu.make_async_copy(
            mout_v.at[s], mask_out.at[pl.ds(base, CHUNK)], osems[s]),
    ]

  def compute(c, s):
    gather_copy(c, s).wait()

    @plsc.parallel_loop(0, NGRP, unroll=2)
    def grp_body(g):
      r0 = c * CHUNK + g * LANES
      rows = r0 + iota
      bloc = lax.shift_right_logical(rows * _DIV20_MUL, _DIV20_SHIFT)
      lpos = rows - bloc * L
      cnt16 = plsc.load_gather(vc_v, [bloc])
      valid = lpos < cnt16
      m16 = jnp.where(valid, 1.0, 0.0).astype(jnp.float32)
      mout_v[s, pl.ds(g * LANES, LANES)] = jnp.where(valid, 0, 1).astype(jnp.int32)
      o0 = g * (LANES * OD)
      for i in range(LANES):
        m = _bcast(m16, i)
        for k in range(D // LANES):
          v = emb_v[s, g * LANES + i, pl.ds(k * LANES, LANES)]
          out_v[s, g * LANES + i, pl.ds(k * LANES, LANES)] = v * m
      for jj in range(4):
        my = _lane_gather(m16, 4 * jj + yaw_row_pat)
        v = yaw_v[pl.ds(r0 * YD + jj * LANES, LANES)]
        rows2 = g * LANES + 4 * jj + yaw_row_pat
        plsc.store_scatter(out_v.at[s], [rows2, yaw_col_pat], v * my)
    for cp in output_copies(c, s):
      cp.start()

  gather_copy(0, 0).start()
  gather_copy(1, 1).start()

  def loop_body(c2, _):
    a = 2 * c2

    @pl.when(c2 > 0)
    def _():
      for cp in output_copies(a - 2, 0):
        cp.wait()
    compute(a, 0)

    @pl.when(c2 < NCHUNK // 2 - 1)
    def _():
      gather_copy(a + 2, 0).start()

    @pl.when(c2 > 0)
    def _():
      for cp in output_copies(a - 1, 1):
        cp.wait()
    compute(a + 1, 1)

    @pl.when(c2 < NCHUNK // 2 - 1)
    def _():
      gather_copy(a + 3, 1).start()
    return 0

  lax.fori_loop(0, NCHUNK // 2, loop_body, 0)
  for cp in output_copies(NCHUNK - 2, 0):
    cp.wait()
  for cp in output_copies(NCHUNK - 1, 1):
    cp.wait()


@jax.jit
def _run(table, idxf, yawf, vc):
  mesh = plsc.VectorSubcoreMesh(core_axis_name="c", subcore_axis_name="s",
                                num_cores=NC, num_subcores=NS)
  f = pl.kernel(
      _body,
      out_type=(
          jax.ShapeDtypeStruct((BL, OD), jnp.float32),
          jax.ShapeDtypeStruct((BL,), jnp.int32),
      ),
      mesh=mesh,
      compiler_params=pltpu.CompilerParams(use_tc_tiling_on_sc=False,
                                           needs_layout_passes=False),
      scratch_types=[
          pltpu.VMEM((ROWS_W,), jnp.int32),
          pltpu.VMEM((ROWS_W * YD,), jnp.float32),
          pltpu.VMEM((BW,), jnp.int32),
          pltpu.VMEM((2, CHUNK), jnp.int32),
          pltpu.VMEM((2, CHUNK, OD), jnp.float32),
          pltpu.VMEM((2, CHUNK, D), jnp.float32),
          pltpu.SemaphoreType.DMA,
          pltpu.SemaphoreType.DMA,
          pltpu.SemaphoreType.DMA,
          pltpu.SemaphoreType.DMA,
          pltpu.SemaphoreType.DMA,
      ],
  )
  return f(table, idxf, yawf, vc)


def kernel(indices, yaw_bits, valid_counts, table):
  idxf = indices.reshape(-1)
  yawf = yaw_bits.reshape(-1)
  feat, mask_i = _run(table, idxf, yawf, valid_counts)
  features = feat.reshape(B, L, OD)
  mask = mask_i.reshape(B, L).astype(bool)
  return features, mask


# speedup vs baseline: 1.2109x; 1.1300x over previous
import jax
import jax.numpy as jnp
from jax import lax
from jax.experimental import pallas as pl
from jax.experimental.pallas import tpu as pltpu
from jax.experimental.pallas import tpu_sc as plsc

B = 16384
L = 20
D = 64
YD = 4
OD = D + YD
BL = B * L

NC = 2
NS = 16
NW = NC * NS
ROWS_W = BL // NW
BW = B // NW
CHUNK = 256
NCHUNK = ROWS_W // CHUNK
LANES = 16
NGRP = CHUNK // LANES

_DIV20_MUL = 52429
_DIV20_SHIFT = 20

_GDN = lax.GatherDimensionNumbers(
    offset_dims=(), collapsed_slice_dims=(0,), start_index_map=(0,))


def _lane_gather(v, idx):
  return lax.gather(v, idx[:, None], _GDN, (1,),
                    mode=lax.GatherScatterMode.PROMISE_IN_BOUNDS)


def _bcast(v, i):
  return _lane_gather(v, jnp.full((LANES,), i, jnp.int32))


def _body(table, idxf, yawf, vc,
          feat_out, mask_out,
          idx_v, yaw_v, vc_v, mout_v, out_v, emb_v,
          gsem0, gsem1, osem0, osem1, bsem):
  wid = lax.axis_index("s") * NC + lax.axis_index("c")
  w0 = wid * ROWS_W

  iota = lax.iota(jnp.int32, LANES)
  yaw_row_pat = lax.shift_right_logical(iota, 2)
  yaw_dst_pat = 68 * yaw_row_pat + D + lax.bitwise_and(iota, 3)

  gsems = (gsem0, gsem1)
  osems = (osem0, osem1)

  pltpu.make_async_copy(idxf.at[pl.ds(w0, ROWS_W)], idx_v, bsem).start()
  pltpu.make_async_copy(yawf.at[pl.ds(w0 * YD, ROWS_W * YD)], yaw_v,
                        bsem).start()
  pltpu.make_async_copy(vc.at[pl.ds(wid * BW, BW)], vc_v, bsem).start()
  pltpu.make_async_copy(idxf.at[pl.ds(w0, ROWS_W)], idx_v, bsem).wait()
  pltpu.make_async_copy(yawf.at[pl.ds(w0 * YD, ROWS_W * YD)], yaw_v,
                        bsem).wait()
  pltpu.make_async_copy(vc.at[pl.ds(wid * BW, BW)], vc_v, bsem).wait()

  def gather_copy(c, s):
    return pltpu.make_async_copy(
        table.at[idx_v.at[pl.ds(c * CHUNK, CHUNK)]],
        emb_v.at[s], gsems[s])

  def output_copies(c, s):
    base = w0 + c * CHUNK
    return [
        pltpu.make_async_copy(
            out_v.at[s], feat_out.at[pl.ds(base * OD, CHUNK * OD)], osems[s]),
        pltpu.make_async_copy(
            mout_v.at[s], mask_out.at[pl.ds(base, CHUNK)], osems[s]),
    ]

  def compute(c, s):
    gather_copy(c, s).wait()

    @plsc.parallel_loop(0, NGRP, unroll=2)
    def grp_body(g):
      r0 = c * CHUNK + g * LANES
      rows = r0 + iota
      bloc = lax.shift_right_logical(rows * _DIV20_MUL, _DIV20_SHIFT)
      lpos = rows - bloc * L
      cnt16 = plsc.load_gather(vc_v, [bloc])
      valid = lpos < cnt16
      m16 = jnp.where(valid, 1.0, 0.0).astype(jnp.float32)
      mout_v[s, pl.ds(g * LANES, LANES)] = jnp.where(valid, 0, 1).astype(jnp.int32)
      o0 = g * (LANES * OD)
      for i in range(LANES):
        m = _bcast(m16, i)
        for k in range(D // LANES):
          v = emb_v[s, g * LANES + i, pl.ds(k * LANES, LANES)]
          out_v[s, pl.ds((g * LANES + i) * OD + k * LANES, LANES)] = v * m
      for jj in range(4):
        my = _lane_gather(m16, 4 * jj + yaw_row_pat)
        v = yaw_v[pl.ds(r0 * YD + jj * LANES, LANES)]
        plsc.store_scatter(out_v.at[s],
                           [g * (LANES * OD) + 272 * jj + yaw_dst_pat], v * my)
    for cp in output_copies(c, s):
      cp.start()

  gather_copy(0, 0).start()
  gather_copy(1, 1).start()

  def loop_body(c2, _):
    a = 2 * c2

    @pl.when(c2 > 0)
    def _():
      for cp in output_copies(a - 2, 0):
        cp.wait()
    compute(a, 0)

    @pl.when(c2 < NCHUNK // 2 - 1)
    def _():
      gather_copy(a + 2, 0).start()

    @pl.when(c2 > 0)
    def _():
      for cp in output_copies(a - 1, 1):
        cp.wait()
    compute(a + 1, 1)

    @pl.when(c2 < NCHUNK // 2 - 1)
    def _():
      gather_copy(a + 3, 1).start()
    return 0

  lax.fori_loop(0, NCHUNK // 2, loop_body, 0)
  for cp in output_copies(NCHUNK - 2, 0):
    cp.wait()
  for cp in output_copies(NCHUNK - 1, 1):
    cp.wait()


@jax.jit
def _run(table, idxf, yawf, vc):
  mesh = plsc.VectorSubcoreMesh(core_axis_name="c", subcore_axis_name="s",
                                num_cores=NC, num_subcores=NS)
  f = pl.kernel(
      _body,
      out_type=(
          jax.ShapeDtypeStruct((BL * OD,), jnp.float32),
          jax.ShapeDtypeStruct((BL,), jnp.int32),
      ),
      mesh=mesh,
      compiler_params=pltpu.CompilerParams(use_tc_tiling_on_sc=False,
                                           needs_layout_passes=False),
      scratch_types=[
          pltpu.VMEM((ROWS_W,), jnp.int32),
          pltpu.VMEM((ROWS_W * YD,), jnp.float32),
          pltpu.VMEM((BW,), jnp.int32),
          pltpu.VMEM((2, CHUNK), jnp.int32),
          pltpu.VMEM((2, CHUNK * OD), jnp.float32),
          pltpu.VMEM((2, CHUNK, D), jnp.float32),
          pltpu.SemaphoreType.DMA,
          pltpu.SemaphoreType.DMA,
          pltpu.SemaphoreType.DMA,
          pltpu.SemaphoreType.DMA,
          pltpu.SemaphoreType.DMA,
      ],
  )
  return f(table, idxf, yawf, vc)


def kernel(indices, yaw_bits, valid_counts, table):
  idxf = indices.reshape(-1)
  yawf = yaw_bits.reshape(-1)
  feat, mask_i = _run(table, idxf, yawf, valid_counts)
  features = feat.reshape(B, L, OD)
  mask = mask_i.reshape(B, L).astype(bool)
  return features, mask
